# baseline (speedup 1.0000x reference)
import jax
import jax.numpy as jnp
from jax import lax
from jax.experimental import pallas as pl
from jax.experimental.pallas import tpu as pltpu
from jax.experimental.pallas import tpu_sc as plsc

_N = 10000
_E = 320000
_D = 128
_R = 8
_NC = 2
_NS = 16
_NW = _NC * _NS
_EPW = _E // _NW
_G = 80
_NG = _EPW // _G
_WPT = 624
_TAIL = _N - _NS * _WPT
_L = 16

_BN = 2000


def _xw_body(x_ref, w_ref, out_ref):
    out_ref[0] = jnp.dot(x_ref[...], w_ref[0], preferred_element_type=jnp.float32)


def _add_body(p_ref, o_ref):
    o_ref[...] = p_ref[0] + p_ref[1]


def _sc_body(xw_hbm, pk_hbm, out_hbm,
             pkv0, pkv1, meta0, meta1, meta2, rows0, rows1, rows2,
             acc,
             lsem0, lsem1, gsem0, gsem1, gsem2, ssem0, ssem1, ssem2):
    c = lax.axis_index("c")
    s = lax.axis_index("s")
    wid = c * _NS + s

    pkv = (pkv0, pkv1)
    meta = (meta0, meta1, meta2)
    rows = (rows0, rows1, rows2)
    lsem = (lsem0, lsem1)
    gsem = (gsem0, gsem1, gsem2)
    ssem = (ssem0, ssem1, ssem2)

    def _zb(i, carry):
        for j in range(_D // _L):
            rows0[i, pl.ds(j * _L, _L)] = jnp.zeros((_L,), jnp.float32)
        return carry
    lax.fori_loop(0, _G, _zb, 0)
    base = pl.multiple_of(s * _WPT, 8)
    nfull = _WPT // _G
    for k in range(nfull):
        pltpu.sync_copy(rows0, acc.at[pl.ds(base + k * _G, _G)])
    rem = _WPT - nfull * _G
    if rem:
        pltpu.sync_copy(rows0.at[pl.ds(0, rem)],
                        acc.at[pl.ds(base + nfull * _G, rem)])

    @pl.when(s == 0)
    def _zero_tail():
        pltpu.sync_copy(rows0.at[pl.ds(0, _TAIL)],
                        acc.at[pl.ds(_NS * _WPT, _TAIL)])
    plsc.subcore_barrier()

    def extract(n, pb):
        for j in range(_G // _L):
            sl = pl.ds(j * _L, _L)
            meta[n][0, sl] = pkv[pb][1, sl] * _N + pkv[pb][0, sl]
            meta[n][1, sl] = pkv[pb][2, sl]
            meta[n][2, sl] = pkv[pb][3, sl]

    def _phase(g, b, pb, s1, s2, s3, w_ssem):
        n = (b + 1) % 3
        if s1:
            pltpu.make_async_copy(pk_hbm.at[wid, 0], pkv[pb], lsem[pb]).wait()
            if w_ssem:
                pltpu.make_async_copy(
                    rows[n], acc.at[meta[n].at[1]], ssem[n]).wait()
            extract(n, pb)
        if s2:
            pltpu.async_copy(pk_hbm.at[wid, g + 3], pkv[pb], lsem[pb])
        if s3:
            pltpu.async_copy(xw_hbm.at[meta[n].at[0]], rows[n], gsem[n])
        pltpu.make_async_copy(
            xw_hbm.at[meta[b].at[0]], rows[b], gsem[b]).wait()

        @plsc.parallel_loop(0, _G, 1, unroll=8)
        def _scale(e):
            ei = jnp.full((_L,), e, jnp.int32)
            ci = jnp.full((_L,), 2, jnp.int32)
            nb = plsc.bitcast(plsc.load_gather(meta[b], [ci, ei]), jnp.float32)
            for j in range(_D // _L):
                sl = pl.ds(j * _L, _L)
                rows[b][e, sl] = rows[b][e, sl] * nb

        pltpu.async_copy(rows[b], acc.at[meta[b].at[1]], ssem[b], add=True)

    pltpu.async_copy(pk_hbm.at[wid, 0], pkv0, lsem0)
    pltpu.async_copy(pk_hbm.at[wid, 1], pkv1, lsem1)
    pltpu.make_async_copy(pk_hbm.at[wid, 0], pkv0, lsem0).wait()
    extract(0, 0)
    pltpu.async_copy(pk_hbm.at[wid, 2], pkv0, lsem0)
    pltpu.async_copy(xw_hbm.at[meta0.at[0]], rows0, gsem0)

    _phase(0, 0, 1, True, True, True, False)
    _phase(1, 1, 0, True, True, True, False)

    def body(k, carry):
        g = 2 + 6 * k
        for i in range(6):
            _phase(g + i, (2 + i) % 3, (3 + i) % 2, True, True, True, True)
        return carry
    lax.fori_loop(0, (_NG - 5) // 6, body, 0)

    _phase(_NG - 3, (_NG - 3) % 3, (_NG - 2) % 2, True, False, True, True)
    _phase(_NG - 2, (_NG - 2) % 3, (_NG - 1) % 2, True, False, True, True)
    _phase(_NG - 1, (_NG - 1) % 3, _NG % 2, False, False, False, False)

    for gg in (_NG - 3, _NG - 2, _NG - 1):
        b = gg % 3
        pltpu.make_async_copy(rows[b], acc.at[meta[b].at[1]], ssem[b]).wait()

    plsc.subcore_barrier()
    wsl = pl.ds(pl.multiple_of(s * _WPT, 8), _WPT)
    pltpu.sync_copy(acc.at[wsl], out_hbm.at[c, wsl])

    @pl.when(s == 0)
    def _write_tail():
        tsl = pl.ds(_NS * _WPT, _TAIL)
        pltpu.sync_copy(acc.at[tsl], out_hbm.at[c, tsl])


def kernel(inputs, edge_index, rel_type, norm, weight):
    xw = pl.pallas_call(
        _xw_body,
        grid=(_N // _BN, _R),
        in_specs=[
            pl.BlockSpec((_BN, _D), lambda i, j: (i, 0)),
            pl.BlockSpec((1, _D, _D), lambda i, j: (j, 0, 0)),
        ],
        out_specs=pl.BlockSpec((1, _BN, _D), lambda i, j: (j, i, 0)),
        out_shape=jax.ShapeDtypeStruct((_R, _N, _D), jnp.float32),
    )(inputs, weight)
    xw_flat = xw.reshape(_R * _N, _D)

    src = edge_index[0].astype(jnp.int32).reshape(_NW, _NG, _G)
    rel = rel_type.astype(jnp.int32).reshape(_NW, _NG, _G)
    dst = edge_index[1].astype(jnp.int32).reshape(_NW, _NG, _G)
    nbits = lax.bitcast_convert_type(
        norm.astype(jnp.float32).reshape(-1), jnp.int32).reshape(_NW, _NG, _G)
    packed = jnp.stack([src, rel, dst, nbits], axis=2)

    mesh = plsc.VectorSubcoreMesh(core_axis_name="c", subcore_axis_name="s")
    partials = pl.kernel(
        _sc_body,
        out_type=jax.ShapeDtypeStruct((_NC, _N, _D), jnp.float32),
        mesh=mesh,
        compiler_params=pltpu.CompilerParams(
            needs_layout_passes=False,
            disable_bounds_checks=True,
            disable_semaphore_checks=True,
        ),
        scratch_types=[
            pltpu.VMEM((4, _G), jnp.int32),
            pltpu.VMEM((4, _G), jnp.int32),
            pltpu.VMEM((3, _G), jnp.int32),
            pltpu.VMEM((3, _G), jnp.int32),
            pltpu.VMEM((3, _G), jnp.int32),
            pltpu.VMEM((_G, _D), jnp.float32),
            pltpu.VMEM((_G, _D), jnp.float32),
            pltpu.VMEM((_G, _D), jnp.float32),
            pltpu.VMEM_SHARED((_N, _D), jnp.float32),
            pltpu.SemaphoreType.DMA,
            pltpu.SemaphoreType.DMA,
            pltpu.SemaphoreType.DMA,
            pltpu.SemaphoreType.DMA,
            pltpu.SemaphoreType.DMA,
            pltpu.SemaphoreType.DMA,
            pltpu.SemaphoreType.DMA,
            pltpu.SemaphoreType.DMA,
        ],
    )(xw_flat, packed)

    h = pl.pallas_call(
        _add_body,
        grid=(_N // _BN,),
        in_specs=[pl.BlockSpec((_NC, _BN, _D), lambda i: (0, i, 0))],
        out_specs=pl.BlockSpec((_BN, _D), lambda i: (i, 0)),
        out_shape=jax.ShapeDtypeStruct((_N, _D), jnp.float32),
    )(partials)
    return h


# speedup vs baseline: 1.1651x; 1.1651x over previous
import jax
import jax.numpy as jnp
from jax import lax
from jax.experimental import pallas as pl
from jax.experimental.pallas import tpu as pltpu
from jax.experimental.pallas import tpu_sc as plsc

_N = 10000
_E = 320000
_D = 128
_R = 8
_NC = 2
_NS = 16
_NW = _NC * _NS
_EPW = _E // _NW
_G = 80
_NG = _EPW // _G
_WPT = 624
_TAIL = _N - _NS * _WPT
_L = 16

_BN = 2000


def _xw_body(x_ref, w_ref, out_ref):
    out_ref[0] = jnp.dot(x_ref[...], w_ref[0], preferred_element_type=jnp.float32)


def _add_body(p_ref, o_ref):
    o_ref[...] = p_ref[0] + p_ref[1]


def _sc_body(xw_hbm, src_hbm, rel_hbm, dst_hbm, nb_hbm, out_hbm,
             pkv0, pkv1, meta0, meta1, meta2, rows0, rows1, rows2,
             acc,
             lsem0, lsem1, gsem0, gsem1, gsem2,
             ssem0, ssem1, ssem2, msem0, msem1, msem2):
    c = lax.axis_index("c")
    s = lax.axis_index("s")
    wid = c * _NS + s
    ebase = wid * _EPW

    pkv = (pkv0, pkv1)
    meta = (meta0, meta1, meta2)
    rows = (rows0, rows1, rows2)
    lsem = (lsem0, lsem1)
    gsem = (gsem0, gsem1, gsem2)
    ssem = (ssem0, ssem1, ssem2)
    msem = (msem0, msem1, msem2)

    def _zb(i, carry):
        for j in range(_D // _L):
            rows0[i, pl.ds(j * _L, _L)] = jnp.zeros((_L,), jnp.float32)
        return carry
    lax.fori_loop(0, _G, _zb, 0)
    base = pl.multiple_of(s * _WPT, 8)
    nfull = _WPT // _G
    for k in range(nfull):
        pltpu.sync_copy(rows0, acc.at[pl.ds(base + k * _G, _G)])
    rem = _WPT - nfull * _G
    if rem:
        pltpu.sync_copy(rows0.at[pl.ds(0, rem)],
                        acc.at[pl.ds(base + nfull * _G, rem)])

    @pl.when(s == 0)
    def _zero_tail():
        pltpu.sync_copy(rows0.at[pl.ds(0, _TAIL)],
                        acc.at[pl.ds(_NS * _WPT, _TAIL)])
    plsc.subcore_barrier()

    def fire_recs(g, pb):
        off = ebase + g * _G
        pltpu.async_copy(src_hbm.at[pl.ds(off, _G)], pkv[pb].at[0], lsem[pb])
        pltpu.async_copy(rel_hbm.at[pl.ds(off, _G)], pkv[pb].at[1], lsem[pb])

    def wait_recs(pb):
        pltpu.make_async_copy(
            src_hbm.at[pl.ds(0, _G)], pkv[pb].at[0], lsem[pb]).wait()
        pltpu.make_async_copy(
            rel_hbm.at[pl.ds(0, _G)], pkv[pb].at[1], lsem[pb]).wait()

    def fire_meta(g, n):
        off = ebase + g * _G
        pltpu.async_copy(dst_hbm.at[pl.ds(off, _G)], meta[n].at[1], msem[n])
        pltpu.async_copy(nb_hbm.at[pl.ds(off, _G)], meta[n].at[2], msem[n])

    def wait_meta(n):
        pltpu.make_async_copy(
            dst_hbm.at[pl.ds(0, _G)], meta[n].at[1], msem[n]).wait()
        pltpu.make_async_copy(
            nb_hbm.at[pl.ds(0, _G)], meta[n].at[2], msem[n]).wait()

    def extract_idx(n, pb):
        for j in range(_G // _L):
            sl = pl.ds(j * _L, _L)
            meta[n][0, sl] = pkv[pb][1, sl] * _N + pkv[pb][0, sl]

    def _phase(g, b, pb, s1, s2, s3, w_ssem):
        n = (b + 1) % 3
        if s1:
            wait_recs(pb)
            if w_ssem:
                pltpu.make_async_copy(
                    rows[n], acc.at[meta[n].at[1]], ssem[n]).wait()
            fire_meta(g + 1, n)
            extract_idx(n, pb)
        if s2:
            fire_recs(g + 3, pb)
        if s3:
            pltpu.async_copy(xw_hbm.at[meta[n].at[0]], rows[n], gsem[n])
        pltpu.make_async_copy(
            xw_hbm.at[meta[b].at[0]], rows[b], gsem[b]).wait()
        wait_meta(b)

        @plsc.parallel_loop(0, _G, 1, unroll=8)
        def _scale(e):
            ei = jnp.full((_L,), e, jnp.int32)
            ci = jnp.full((_L,), 2, jnp.int32)
            nb = plsc.bitcast(plsc.load_gather(meta[b], [ci, ei]), jnp.float32)
            for j in range(_D // _L):
                sl = pl.ds(j * _L, _L)
                rows[b][e, sl] = rows[b][e, sl] * nb

        pltpu.async_copy(rows[b], acc.at[meta[b].at[1]], ssem[b], add=True)

    fire_recs(0, 0)
    fire_recs(1, 1)
    fire_meta(0, 0)
    wait_recs(0)
    extract_idx(0, 0)
    fire_recs(2, 0)
    pltpu.async_copy(xw_hbm.at[meta0.at[0]], rows0, gsem0)

    _phase(0, 0, 1, True, True, True, False)
    _phase(1, 1, 0, True, True, True, False)

    def body(k, carry):
        g = 2 + 6 * k
        for i in range(6):
            _phase(g + i, (2 + i) % 3, (3 + i) % 2, True, True, True, True)
        return carry
    lax.fori_loop(0, (_NG - 5) // 6, body, 0)

    _phase(_NG - 3, (_NG - 3) % 3, (_NG - 2) % 2, True, False, True, True)
    _phase(_NG - 2, (_NG - 2) % 3, (_NG - 1) % 2, True, False, True, True)
    _phase(_NG - 1, (_NG - 1) % 3, _NG % 2, False, False, False, False)

    for gg in (_NG - 3, _NG - 2, _NG - 1):
        b = gg % 3
        pltpu.make_async_copy(rows[b], acc.at[meta[b].at[1]], ssem[b]).wait()

    plsc.subcore_barrier()
    wsl = pl.ds(pl.multiple_of(s * _WPT, 8), _WPT)
    pltpu.sync_copy(acc.at[wsl], out_hbm.at[c, wsl])

    @pl.when(s == 0)
    def _write_tail():
        tsl = pl.ds(_NS * _WPT, _TAIL)
        pltpu.sync_copy(acc.at[tsl], out_hbm.at[c, tsl])


def kernel(inputs, edge_index, rel_type, norm, weight):
    xw = pl.pallas_call(
        _xw_body,
        grid=(_N // _BN, _R),
        in_specs=[
            pl.BlockSpec((_BN, _D), lambda i, j: (i, 0)),
            pl.BlockSpec((1, _D, _D), lambda i, j: (j, 0, 0)),
        ],
        out_specs=pl.BlockSpec((1, _BN, _D), lambda i, j: (j, i, 0)),
        out_shape=jax.ShapeDtypeStruct((_R, _N, _D), jnp.float32),
    )(inputs, weight)
    xw_flat = xw.reshape(_R * _N, _D)

    src = edge_index[0].astype(jnp.int32).reshape(_E)
    rel = rel_type.astype(jnp.int32).reshape(_E)
    dst = edge_index[1].astype(jnp.int32).reshape(_E)
    nbits = lax.bitcast_convert_type(norm.astype(jnp.float32).reshape(_E),
                                     jnp.int32)

    mesh = plsc.VectorSubcoreMesh(core_axis_name="c", subcore_axis_name="s")
    partials = pl.kernel(
        _sc_body,
        out_type=jax.ShapeDtypeStruct((_NC, _N, _D), jnp.float32),
        mesh=mesh,
        compiler_params=pltpu.CompilerParams(needs_layout_passes=False),
        scratch_types=[
            pltpu.VMEM((2, _G), jnp.int32),
            pltpu.VMEM((2, _G), jnp.int32),
            pltpu.VMEM((3, _G), jnp.int32),
            pltpu.VMEM((3, _G), jnp.int32),
            pltpu.VMEM((3, _G), jnp.int32),
            pltpu.VMEM((_G, _D), jnp.float32),
            pltpu.VMEM((_G, _D), jnp.float32),
            pltpu.VMEM((_G, _D), jnp.float32),
            pltpu.VMEM_SHARED((_N, _D), jnp.float32),
            pltpu.SemaphoreType.DMA,
            pltpu.SemaphoreType.DMA,
            pltpu.SemaphoreType.DMA,
            pltpu.SemaphoreType.DMA,
            pltpu.SemaphoreType.DMA,
            pltpu.SemaphoreType.DMA,
            pltpu.SemaphoreType.DMA,
            pltpu.SemaphoreType.DMA,
            pltpu.SemaphoreType.DMA,
            pltpu.SemaphoreType.DMA,
            pltpu.SemaphoreType.DMA,
        ],
    )(xw_flat, src, rel, dst, nbits)

    h = pl.pallas_call(
        _add_body,
        grid=(_N // _BN,),
        in_specs=[pl.BlockSpec((_NC, _BN, _D), lambda i: (0, i, 0))],
        out_specs=pl.BlockSpec((_BN, _D), lambda i: (i, 0)),
        out_shape=jax.ShapeDtypeStruct((_N, _D), jnp.float32),
    )(partials)
    return h
